# baseline (speedup 1.0000x reference)
import functools

import jax
import jax.numpy as jnp
import numpy as np
from jax import lax
from jax.experimental import pallas as pl
from jax.experimental.pallas import tpu as pltpu
from jax.experimental.pallas import tpu_sc as plsc

_S, _D, _H = 2048, 64, 16
_P, _U, _G = 32, 12, 4
_F, _KK = 16, 6
_TAU, _ALPHA = 8.0, 0.2
_SCALE = 1.0 / np.sqrt(_D)
_IDXP = np.round(np.linspace(0.0, _S - 1, _P)).astype(np.int32)
_NEG = -1e30
_C = 256
_CH = _C + 16
_NCH = _S // _C
_KBS = [int(np.clip(c * _C - 8, 0, _S - _CH)) for c in range(_NCH)]


def _globals_tc_body(k_ref, q_ref, v_ref, og_ref, mz_ref):
    k2 = k_ref[0]
    q2 = q_ref[0]
    v2 = v_ref[0]
    eps = 1e-6
    kn = jnp.sqrt(jnp.sum(k2 * k2, -1, keepdims=True))
    Kbar = k2 / jnp.maximum(kn, eps)
    qn = jnp.sqrt(jnp.sum(q2 * q2, -1, keepdims=True))
    qs = q2 / jnp.maximum(qn, eps)
    Qp = jnp.concatenate([qs[int(i)][None] for i in _IDXP], axis=0)
    Smat = jnp.maximum(
        jax.lax.dot_general(Kbar.astype(jnp.bfloat16), Qp.astype(jnp.bfloat16),
                            (((1,), (1,)), ((), ())),
                            preferred_element_type=jnp.float32), 0.0)
    mean = jnp.mean(Smat, -1, keepdims=True)
    mx = jnp.max(Smat, -1, keepdims=True)
    iota_p = jax.lax.broadcasted_iota(jnp.int32, (_S, _P), 1)
    cur = Smat
    acc = jnp.zeros((_S, 1), jnp.float32)
    for _ in range(6):
        m = jnp.max(cur, -1, keepdims=True)
        pos = jnp.min(jnp.where(cur == m, iota_p, _P), -1, keepdims=True)
        acc = acc + m
        cur = jnp.where(iota_p == pos, _NEG, cur)
    topk_mean = acc / 6.0
    std = jnp.sqrt(jnp.sum((Smat - mean) ** 2, -1, keepdims=True) / (_P - 1))
    u = 1.0 * mean + 0.6 * mx + 0.4 * topk_mean + 0.2 * std
    iota_s = jax.lax.broadcasted_iota(jnp.int32, (_S, 1), 0)
    pos_rows = []
    sub_rows = []
    uu = u
    for _ in range(_U):
        m = jnp.max(uu)
        pos = jnp.min(jnp.where(uu == m, iota_s, _S))
        oh = (iota_s == pos).astype(jnp.float32)
        sub_rows.append(jnp.sum(oh * Smat, axis=0, keepdims=True))
        pos_rows.append(pos)
        uu = jnp.where(iota_s == pos, _NEG, uu)
    S_sub = jnp.concatenate(sub_rows, axis=0)
    iota_u = jax.lax.broadcasted_iota(jnp.int32, (_U, 1), 0)
    top_pos = jnp.zeros((_U, 1), jnp.int32)
    for r, pv in enumerate(pos_rows):
        top_pos = jnp.where(iota_u == r, pv, top_pos)
    mvec = jnp.zeros((1, _P), jnp.float32)
    blocked = jnp.zeros((_U, 1), jnp.bool_)
    chosen = []
    for _ in range(_G):
        gains = jnp.sum(jnp.maximum(S_sub - mvec, 0.0), -1, keepdims=True)
        gains = jnp.where(blocked, -1e9, gains)
        gm = jnp.max(gains)
        j = jnp.min(jnp.where(gains == gm, iota_u, _U))
        ohj = iota_u == j
        chosen.append(jnp.sum(jnp.where(ohj, top_pos, 0)))
        blocked = blocked | ohj
        mvec = jnp.maximum(mvec, jnp.sum(jnp.where(ohj, S_sub, 0.0),
                                         axis=0, keepdims=True))
    kg_rows, vg_rows = [], []
    for c in chosen:
        oh = (iota_s == c).astype(jnp.float32)
        kg_rows.append(jnp.sum(oh * k2, axis=0, keepdims=True))
        vg_rows.append(jnp.sum(oh * v2, axis=0, keepdims=True))
    kg = jnp.concatenate(kg_rows, 0)
    vg = jnp.concatenate(vg_rows, 0)
    lgT = jax.lax.dot_general(kg.astype(jnp.bfloat16), q2.astype(jnp.bfloat16),
                              (((1,), (1,)), ((), ())),
                              preferred_element_type=jnp.float32) * _SCALE
    Mg = jnp.max(lgT, 0, keepdims=True)
    egT = jnp.exp(lgT - Mg)
    Zg = jnp.sum(egT, 0, keepdims=True)
    Og = jax.lax.dot_general(egT, vg, (((0,), (0,)), ((), ())))
    og_ref[0] = Og
    mz_ref[0] = jnp.concatenate([Mg, Zg], axis=0)


def _globals_stage(k3, q3, v3):
    return pl.pallas_call(
        _globals_tc_body,
        grid=(_H,),
        in_specs=[pl.BlockSpec((1, _S, _D), lambda h: (h, 0, 0))] * 3,
        out_specs=[pl.BlockSpec((1, _S, _D), lambda h: (h, 0, 0)),
                   pl.BlockSpec((1, 2, _S), lambda h: (h, 0, 0))],
        out_shape=[jax.ShapeDtypeStruct((_H, _S, _D), jnp.float32),
                   jax.ShapeDtypeStruct((_H, 2, _S), jnp.float32)],
    )(k3, q3, v3)


_mesh = plsc.VectorSubcoreMesh(core_axis_name="c", subcore_axis_name="s")


@functools.partial(
    pl.kernel,
    mesh=_mesh,
    compiler_params=pltpu.CompilerParams(needs_layout_passes=False,
                                         use_tc_tiling_on_sc=False),
    out_type=jax.ShapeDtypeStruct((_H, _S * _D), jnp.float32),
    scratch_types=[
        pltpu.VMEM((_C * _D,), jnp.float32),
        pltpu.VMEM((_CH * _D,), jnp.float32),
        pltpu.VMEM((_CH * _D,), jnp.float32),
        pltpu.VMEM((_C * _D,), jnp.float32),
        pltpu.VMEM((2, _C), jnp.float32),
        pltpu.VMEM((_C * _D,), jnp.float32),
    ],
)
def _local_sc(qf, kf, vf, ogf, mz, outf, q_v, k_v, v_v, og_v, mz_v, out_v):
    core = lax.axis_index("c")
    head = lax.axis_index("s")
    lane = jnp.arange(_F, dtype=jnp.int32)

    def chunk_body(cc, _):
        c_abs = core * (_NCH // 2) + cc
        c0 = c_abs * _C
        kb = jnp.clip(c0 - 8, 0, _S - _CH)
        pltpu.sync_copy(qf.at[head, pl.ds(c0 * _D, _C * _D)], q_v)
        pltpu.sync_copy(kf.at[head, pl.ds(kb * _D, _CH * _D)], k_v)
        pltpu.sync_copy(vf.at[head, pl.ds(kb * _D, _CH * _D)], v_v)
        pltpu.sync_copy(ogf.at[head, pl.ds(c0 * _D, _C * _D)], og_v)
        pltpu.sync_copy(mz.at[head, 0, pl.ds(c0, _C)], mz_v.at[0])
        pltpu.sync_copy(mz.at[head, 1, pl.ds(c0, _C)], mz_v.at[1])

        def group_body(gi, _g):
            soff = gi * _F
            qpos = c0 + soff + lane
            srow = jnp.clip(qpos - 8, 0, _S - _F)
            qbase = (soff + lane) * _D
            idxs = [(srow - kb + f) * _D for f in range(_F)]

            def score_d(d, accs):
                qv = plsc.load_gather(q_v, [qbase + d])
                return tuple(
                    accs[f] + qv * plsc.load_gather(k_v, [idxs[f] + d])
                    for f in range(_F))

            accs = lax.fori_loop(
                0, _D, score_d,
                tuple(jnp.zeros((_F,), jnp.float32) for _ in range(_F)))
            scores = [a * _SCALE for a in accs]
            sel = []
            for f in range(_F):
                dist = jnp.abs(srow + f - qpos).astype(jnp.float32)
                sel.append(scores[f] + _ALPHA * jnp.exp(dist * (-1.0 / _TAU)))
            rank = [jnp.zeros((_F,), jnp.float32) for _ in range(_F)]
            for lo in range(_F):
                for hi in range(lo + 1, _F):
                    ge = jnp.where(sel[lo] >= sel[hi], 1.0, 0.0)
                    rank[hi] = rank[hi] + ge
                    rank[lo] = rank[lo] + (1.0 - ge)
            mloc = scores[0]
            for f in range(1, _F):
                mloc = jnp.maximum(mloc, scores[f])
            es = [jnp.where(rank[f] < float(_KK),
                            jnp.exp(scores[f] - mloc), 0.0)
                  for f in range(_F)]
            z = es[0]
            for f in range(1, _F):
                z = z + es[f]
            mg = mz_v[0, pl.ds(soff, _F)]
            zg = mz_v[1, pl.ds(soff, _F)]
            mfull = jnp.maximum(mloc, mg)
            clc = jnp.exp(mloc - mfull)
            cgl = jnp.exp(mg - mfull)
            inv = 1.0 / (clc * z + cgl * zg)
            ws = [e * (clc * inv) for e in es]
            bg = cgl * inv

            def out_d(d, _o):
                acc = bg * plsc.load_gather(og_v, [qbase + d])
                for f in range(_F):
                    acc = acc + ws[f] * plsc.load_gather(v_v, [idxs[f] + d])
                plsc.store_scatter(out_v, [qbase + d], acc)
                return 0

            lax.fori_loop(0, _D, out_d, 0)
            return 0

        lax.fori_loop(0, _C // _F, group_body, 0)
        pltpu.sync_copy(out_v, outf.at[head, pl.ds(c0 * _D, _C * _D)])
        return 0

    lax.fori_loop(0, _NCH // 2, chunk_body, 0)


def _round_bf16(x):
    u = jax.lax.bitcast_convert_type(x, jnp.uint32)
    bias = jnp.uint32(0x7FFF) + ((u >> 16) & jnp.uint32(1))
    return jax.lax.bitcast_convert_type((u + bias) & jnp.uint32(0xFFFF0000),
                                        jnp.float32)


def kernel(q, k, v):
    q3, k3, v3 = q[0], k[0], v[0]
    og, mz = _globals_stage(k3, q3, v3)
    qr = _round_bf16(q3).reshape(_H, _S * _D)
    kr = _round_bf16(k3).reshape(_H, _S * _D)
    vfl = v3.reshape(_H, _S * _D)
    ogf = og.reshape(_H, _S * _D)
    outf = _local_sc(qr, kr, vfl, ogf, mz)
    return outf.reshape(_H, _S, _D)[None]


# speedup vs baseline: 2.9971x; 2.9971x over previous
import functools

import jax
import jax.numpy as jnp
import numpy as np
from jax import lax
from jax.experimental import pallas as pl
from jax.experimental.pallas import tpu as pltpu
from jax.experimental.pallas import tpu_sc as plsc

_S, _D, _H = 2048, 64, 16
_P, _U, _G = 32, 12, 4
_F, _KK = 16, 6
_TAU, _ALPHA = 8.0, 0.2
_SCALE = 1.0 / np.sqrt(_D)
_IDXP = np.round(np.linspace(0.0, _S - 1, _P)).astype(np.int32)
_NEG = -1e30
_C = 256
_CH = _C + 16
_NCH = _S // _C
_KBS = [int(np.clip(c * _C - 8, 0, _S - _CH)) for c in range(_NCH)]


def _globals_tc_body(k_ref, q_ref, v_ref, og_ref, mz_ref):
    k2 = k_ref[0]
    q2 = q_ref[0]
    v2 = v_ref[0]
    eps = 1e-6
    kn = jnp.sqrt(jnp.sum(k2 * k2, -1, keepdims=True))
    Kbar = k2 / jnp.maximum(kn, eps)
    qn = jnp.sqrt(jnp.sum(q2 * q2, -1, keepdims=True))
    qs = q2 / jnp.maximum(qn, eps)
    Qp = jnp.concatenate([qs[int(i)][None] for i in _IDXP], axis=0)
    Smat = jnp.maximum(
        jax.lax.dot_general(Qp.astype(jnp.bfloat16), Kbar.astype(jnp.bfloat16),
                            (((1,), (1,)), ((), ())),
                            preferred_element_type=jnp.float32), 0.0)
    mean = jnp.mean(Smat, 0, keepdims=True)
    mx = jnp.max(Smat, 0, keepdims=True)
    iota_p = jax.lax.broadcasted_iota(jnp.int32, (_P, _S), 0)
    cur = Smat
    acc = jnp.zeros((1, _S), jnp.float32)
    for _ in range(6):
        m = jnp.max(cur, 0, keepdims=True)
        pos = jnp.min(jnp.where(cur == m, iota_p, _P), 0, keepdims=True)
        acc = acc + m
        cur = jnp.where(iota_p == pos, _NEG, cur)
    topk_mean = acc / 6.0
    std = jnp.sqrt(jnp.sum((Smat - mean) ** 2, 0, keepdims=True) / (_P - 1))
    u = 1.0 * mean + 0.6 * mx + 0.4 * topk_mean + 0.2 * std
    iota_s = jax.lax.broadcasted_iota(jnp.int32, (1, _S), 1)
    pos_rows = []
    sub_cols = []
    uu = u
    for _ in range(_U):
        m = jnp.max(uu)
        pos = jnp.min(jnp.where(uu == m, iota_s, _S))
        ohf = (iota_s == pos).astype(jnp.float32)
        sub_cols.append(jnp.sum(ohf * Smat, axis=1, keepdims=True))
        pos_rows.append(pos)
        uu = jnp.where(iota_s == pos, _NEG, uu)
    S_sub = jnp.concatenate(sub_cols, axis=1)
    iota_u = jax.lax.broadcasted_iota(jnp.int32, (1, _U), 1)
    top_pos = jnp.zeros((1, _U), jnp.int32)
    for r, pv in enumerate(pos_rows):
        top_pos = jnp.where(iota_u == r, pv, top_pos)
    mvec = jnp.zeros((_P, 1), jnp.float32)
    blocked = jnp.zeros((1, _U), jnp.bool_)
    chosen = []
    for _ in range(_G):
        gains = jnp.sum(jnp.maximum(S_sub - mvec, 0.0), 0, keepdims=True)
        gains = jnp.where(blocked, -1e9, gains)
        gm = jnp.max(gains)
        j = jnp.min(jnp.where(gains == gm, iota_u, _U))
        ohj = iota_u == j
        chosen.append(jnp.sum(jnp.where(ohj, top_pos, 0)))
        blocked = blocked | ohj
        mvec = jnp.maximum(mvec, jnp.sum(jnp.where(ohj, S_sub, 0.0),
                                         axis=1, keepdims=True))
    iota_sc = jax.lax.broadcasted_iota(jnp.int32, (_S, 1), 0)
    kg_rows, vg_rows = [], []
    for c in chosen:
        oh = (iota_sc == c).astype(jnp.float32)
        kg_rows.append(jnp.sum(oh * k2, axis=0, keepdims=True))
        vg_rows.append(jnp.sum(oh * v2, axis=0, keepdims=True))
    kg = jnp.concatenate(kg_rows, 0)
    vg = jnp.concatenate(vg_rows, 0)
    lgT = jax.lax.dot_general(kg.astype(jnp.bfloat16), q2.astype(jnp.bfloat16),
                              (((1,), (1,)), ((), ())),
                              preferred_element_type=jnp.float32) * _SCALE
    Mg = jnp.max(lgT, 0, keepdims=True)
    egT = jnp.exp(lgT - Mg)
    Zg = jnp.sum(egT, 0, keepdims=True)
    OgT = jax.lax.dot_general(vg, egT, (((0,), (0,)), ((), ())))
    og_ref[0] = OgT
    mz_ref[0] = jnp.concatenate([Mg, Zg], axis=0)


def _globals_stage(k3, q3, v3):
    return pl.pallas_call(
        _globals_tc_body,
        grid=(_H,),
        in_specs=[pl.BlockSpec((1, _S, _D), lambda h: (h, 0, 0))] * 3,
        out_specs=[pl.BlockSpec((1, _D, _S), lambda h: (h, 0, 0)),
                   pl.BlockSpec((1, 2, _S), lambda h: (h, 0, 0))],
        out_shape=[jax.ShapeDtypeStruct((_H, _D, _S), jnp.float32),
                   jax.ShapeDtypeStruct((_H, 2, _S), jnp.float32)],
    )(k3, q3, v3)


_mesh = plsc.VectorSubcoreMesh(core_axis_name="c", subcore_axis_name="s")


@functools.partial(
    pl.kernel,
    mesh=_mesh,
    compiler_params=pltpu.CompilerParams(needs_layout_passes=False),
    out_type=jax.ShapeDtypeStruct((_H, _NCH, _D, _C), jnp.float32),
    scratch_types=[
        pltpu.VMEM((_D, _C), jnp.float32),
        pltpu.VMEM((_D * _CH,), jnp.float32),
        pltpu.VMEM((_D * _CH,), jnp.float32),
        pltpu.VMEM((_D, _C), jnp.float32),
        pltpu.VMEM((2, _C), jnp.float32),
        pltpu.VMEM((_D, _C), jnp.float32),
    ],
)
def _local_sc(qc, kc, vc, ogc, mzc, outc, q_v, k_v, v_v, og_v, mz_v, out_v):
    core = lax.axis_index("c")
    head = lax.axis_index("s")
    lane = jnp.arange(_F, dtype=jnp.int32)

    def chunk_body(cc, _):
        c_abs = core * (_NCH // 2) + cc
        c0 = c_abs * _C
        kb = jnp.clip(c0 - 8, 0, _S - _CH)
        pltpu.sync_copy(qc.at[head, c_abs], q_v)
        pltpu.sync_copy(kc.at[head, c_abs], k_v)
        pltpu.sync_copy(vc.at[head, c_abs], v_v)
        pltpu.sync_copy(ogc.at[head, c_abs], og_v)
        pltpu.sync_copy(mzc.at[head, c_abs], mz_v)

        def group_body(gi, _g):
            soff = gi * _F
            qpos = c0 + soff + lane
            srow = jnp.clip(qpos - 8, 0, _S - _F)
            base = srow - kb
            idxs = [base + f for f in range(_F)]

            def score_d(d, accs):
                drow = d * _CH
                qv = q_v[d, pl.ds(soff, _F)]
                return tuple(
                    accs[f] + qv * plsc.load_gather(k_v, [idxs[f] + drow])
                    for f in range(_F))

            accs = lax.fori_loop(
                0, _D, score_d,
                tuple(jnp.zeros((_F,), jnp.float32) for _ in range(_F)))
            scores = [a * _SCALE for a in accs]
            sel = []
            for f in range(_F):
                dist = jnp.abs(srow + f - qpos).astype(jnp.float32)
                sel.append(scores[f] + _ALPHA * jnp.exp(dist * (-1.0 / _TAU)))
            rank = [jnp.zeros((_F,), jnp.float32) for _ in range(_F)]
            for lo in range(_F):
                for hi in range(lo + 1, _F):
                    ge = jnp.where(sel[lo] >= sel[hi], 1.0, 0.0)
                    rank[hi] = rank[hi] + ge
                    rank[lo] = rank[lo] + (1.0 - ge)
            mloc = scores[0]
            for f in range(1, _F):
                mloc = jnp.maximum(mloc, scores[f])
            es = [jnp.where(rank[f] < float(_KK),
                            jnp.exp(scores[f] - mloc), 0.0)
                  for f in range(_F)]
            z = es[0]
            for f in range(1, _F):
                z = z + es[f]
            mg = mz_v[0, pl.ds(soff, _F)]
            zg = mz_v[1, pl.ds(soff, _F)]
            mfull = jnp.maximum(mloc, mg)
            clc = jnp.exp(mloc - mfull)
            cgl = jnp.exp(mg - mfull)
            inv = 1.0 / (clc * z + cgl * zg)
            ws = [e * (clc * inv) for e in es]
            bg = cgl * inv

            def out_d(d, _o):
                drow = d * _CH
                acc = bg * og_v[d, pl.ds(soff, _F)]
                for f in range(_F):
                    acc = acc + ws[f] * plsc.load_gather(v_v, [idxs[f] + drow])
                out_v[d, pl.ds(soff, _F)] = acc
                return 0

            lax.fori_loop(0, _D, out_d, 0)
            return 0

        lax.fori_loop(0, _C // _F, group_body, 0)
        pltpu.sync_copy(out_v, outc.at[head, c_abs])
        return 0

    lax.fori_loop(0, _NCH // 2, chunk_body, 0)


def _round_bf16(x):
    u = jax.lax.bitcast_convert_type(x, jnp.uint32)
    bias = jnp.uint32(0x7FFF) + ((u >> 16) & jnp.uint32(1))
    return jax.lax.bitcast_convert_type((u + bias) & jnp.uint32(0xFFFF0000),
                                        jnp.float32)


def kernel(q, k, v):
    q3, k3, v3 = q[0], k[0], v[0]
    og, mz = _globals_stage(k3, q3, v3)
    qr = _round_bf16(q3)
    kr = _round_bf16(k3)
    qT = qr.transpose(0, 2, 1)
    kT = kr.transpose(0, 2, 1)
    vT = v3.transpose(0, 2, 1)
    qc = qT.reshape(_H, _D, _NCH, _C).transpose(0, 2, 1, 3)
    kc = jnp.stack([kT[:, :, kb:kb + _CH] for kb in _KBS],
                   axis=1).reshape(_H, _NCH, _D * _CH)
    vc = jnp.stack([vT[:, :, kb:kb + _CH] for kb in _KBS],
                   axis=1).reshape(_H, _NCH, _D * _CH)
    ogc = og.reshape(_H, _D, _NCH, _C).transpose(0, 2, 1, 3)
    mzc = mz.reshape(_H, 2, _NCH, _C).transpose(0, 2, 1, 3)
    outc = _local_sc(qc, kc, vc, ogc, mzc)
    outT = outc.transpose(0, 2, 1, 3).reshape(_H, _D, _S)
    return outT.transpose(0, 2, 1)[None]


# speedup vs baseline: 3.0459x; 1.0163x over previous
import functools

import jax
import jax.numpy as jnp
import numpy as np
from jax import lax
from jax.experimental import pallas as pl
from jax.experimental.pallas import tpu as pltpu
from jax.experimental.pallas import tpu_sc as plsc

_S, _D, _H = 2048, 64, 16
_P, _U, _G = 32, 12, 4
_F, _KK = 16, 6
_TAU, _ALPHA = 8.0, 0.2
_SCALE = 1.0 / np.sqrt(_D)
_IDXP = np.round(np.linspace(0.0, _S - 1, _P)).astype(np.int32)
_NEG = -1e30
_C = 256
_CH = _C + 16
_NCH = _S // _C
_KBS = [int(np.clip(c * _C - 8, 0, _S - _CH)) for c in range(_NCH)]


def _globals_tc_body(k_ref, q_ref, v_ref, og_ref, mz_ref,
                     qt_ref, kt_ref, vt_ref):
    k2 = k_ref[0]
    q2 = q_ref[0]
    v2 = v_ref[0]
    qt_ref[0] = jnp.transpose(_round_bf16(q2), (1, 0))
    kt_ref[0] = jnp.transpose(_round_bf16(k2), (1, 0))
    vt_ref[0] = jnp.transpose(v2, (1, 0))
    eps = 1e-6
    kn = jnp.sqrt(jnp.sum(k2 * k2, -1, keepdims=True))
    Kbar = k2 / jnp.maximum(kn, eps)
    qn = jnp.sqrt(jnp.sum(q2 * q2, -1, keepdims=True))
    qs = q2 / jnp.maximum(qn, eps)
    Qp = jnp.concatenate([qs[int(i)][None] for i in _IDXP], axis=0)
    Smat = jnp.maximum(
        jax.lax.dot_general(Qp.astype(jnp.bfloat16), Kbar.astype(jnp.bfloat16),
                            (((1,), (1,)), ((), ())),
                            preferred_element_type=jnp.float32), 0.0)
    mean = jnp.mean(Smat, 0, keepdims=True)
    mx = jnp.max(Smat, 0, keepdims=True)
    iota_p = jax.lax.broadcasted_iota(jnp.int32, (_P, _S), 0)
    cur = Smat
    acc = jnp.zeros((1, _S), jnp.float32)
    for _ in range(6):
        m = jnp.max(cur, 0, keepdims=True)
        pos = jnp.min(jnp.where(cur == m, iota_p, _P), 0, keepdims=True)
        acc = acc + m
        cur = jnp.where(iota_p == pos, _NEG, cur)
    topk_mean = acc / 6.0
    std = jnp.sqrt(jnp.sum((Smat - mean) ** 2, 0, keepdims=True) / (_P - 1))
    u = 1.0 * mean + 0.6 * mx + 0.4 * topk_mean + 0.2 * std
    iota_s = jax.lax.broadcasted_iota(jnp.int32, (1, _S), 1)
    pos_rows = []
    sub_cols = []
    uu = u
    for _ in range(_U):
        m = jnp.max(uu)
        pos = jnp.min(jnp.where(uu == m, iota_s, _S))
        ohf = (iota_s == pos).astype(jnp.float32)
        sub_cols.append(jnp.sum(ohf * Smat, axis=1, keepdims=True))
        pos_rows.append(pos)
        uu = jnp.where(iota_s == pos, _NEG, uu)
    S_sub = jnp.concatenate(sub_cols, axis=1)
    iota_u = jax.lax.broadcasted_iota(jnp.int32, (1, _U), 1)
    top_pos = jnp.zeros((1, _U), jnp.int32)
    for r, pv in enumerate(pos_rows):
        top_pos = jnp.where(iota_u == r, pv, top_pos)
    mvec = jnp.zeros((_P, 1), jnp.float32)
    blocked = jnp.zeros((1, _U), jnp.bool_)
    chosen = []
    for _ in range(_G):
        gains = jnp.sum(jnp.maximum(S_sub - mvec, 0.0), 0, keepdims=True)
        gains = jnp.where(blocked, -1e9, gains)
        gm = jnp.max(gains)
        j = jnp.min(jnp.where(gains == gm, iota_u, _U))
        ohj = iota_u == j
        chosen.append(jnp.sum(jnp.where(ohj, top_pos, 0)))
        blocked = blocked | ohj
        mvec = jnp.maximum(mvec, jnp.sum(jnp.where(ohj, S_sub, 0.0),
                                         axis=1, keepdims=True))
    iota_sc = jax.lax.broadcasted_iota(jnp.int32, (_S, 1), 0)
    kg_rows, vg_rows = [], []
    for c in chosen:
        oh = (iota_sc == c).astype(jnp.float32)
        kg_rows.append(jnp.sum(oh * k2, axis=0, keepdims=True))
        vg_rows.append(jnp.sum(oh * v2, axis=0, keepdims=True))
    kg = jnp.concatenate(kg_rows, 0)
    vg = jnp.concatenate(vg_rows, 0)
    lgT = jax.lax.dot_general(kg.astype(jnp.bfloat16), q2.astype(jnp.bfloat16),
                              (((1,), (1,)), ((), ())),
                              preferred_element_type=jnp.float32) * _SCALE
    Mg = jnp.max(lgT, 0, keepdims=True)
    egT = jnp.exp(lgT - Mg)
    Zg = jnp.sum(egT, 0, keepdims=True)
    OgT = jax.lax.dot_general(vg, egT, (((0,), (0,)), ((), ())))
    og_ref[0] = OgT
    mz_ref[0] = jnp.concatenate([Mg, Zg], axis=0)


def _globals_stage(k3, q3, v3):
    return pl.pallas_call(
        _globals_tc_body,
        grid=(_H,),
        in_specs=[pl.BlockSpec((1, _S, _D), lambda h: (h, 0, 0))] * 3,
        out_specs=[pl.BlockSpec((1, _D, _S), lambda h: (h, 0, 0)),
                   pl.BlockSpec((1, 2, _S), lambda h: (h, 0, 0)),
                   pl.BlockSpec((1, _D, _S), lambda h: (h, 0, 0)),
                   pl.BlockSpec((1, _D, _S), lambda h: (h, 0, 0)),
                   pl.BlockSpec((1, _D, _S), lambda h: (h, 0, 0))],
        out_shape=[jax.ShapeDtypeStruct((_H, _D, _S), jnp.float32),
                   jax.ShapeDtypeStruct((_H, 2, _S), jnp.float32),
                   jax.ShapeDtypeStruct((_H, _D, _S), jnp.float32),
                   jax.ShapeDtypeStruct((_H, _D, _S), jnp.float32),
                   jax.ShapeDtypeStruct((_H, _D, _S), jnp.float32)],
    )(k3, q3, v3)


_mesh = plsc.VectorSubcoreMesh(core_axis_name="c", subcore_axis_name="s")


@functools.partial(
    pl.kernel,
    mesh=_mesh,
    compiler_params=pltpu.CompilerParams(needs_layout_passes=False),
    out_type=jax.ShapeDtypeStruct((_H, _NCH, _D, _C), jnp.float32),
    scratch_types=[
        pltpu.VMEM((_D, _C), jnp.float32),
        pltpu.VMEM((_D * _CH,), jnp.float32),
        pltpu.VMEM((_D * _CH,), jnp.float32),
        pltpu.VMEM((_D, _C), jnp.float32),
        pltpu.VMEM((2, _C), jnp.float32),
        pltpu.VMEM((_D, _C), jnp.float32),
    ],
)
def _local_sc(qc, kc, vc, ogc, mzc, outc, q_v, k_v, v_v, og_v, mz_v, out_v):
    core = lax.axis_index("c")
    head = lax.axis_index("s")
    lane = jnp.arange(_F, dtype=jnp.int32)

    def chunk_body(cc, _):
        c_abs = core * (_NCH // 2) + cc
        c0 = c_abs * _C
        kb = jnp.clip(c0 - 8, 0, _S - _CH)
        pltpu.sync_copy(qc.at[head, c_abs], q_v)
        pltpu.sync_copy(kc.at[head, c_abs], k_v)
        pltpu.sync_copy(vc.at[head, c_abs], v_v)
        pltpu.sync_copy(ogc.at[head, c_abs], og_v)
        pltpu.sync_copy(mzc.at[head, c_abs], mz_v)

        def group_body(gi, _g):
            soff = gi * _F
            qpos = c0 + soff + lane
            srow = jnp.clip(qpos - 8, 0, _S - _F)
            base = srow - kb
            idxs = [base + f for f in range(_F)]

            def score_d(d, accs):
                drow = d * _CH
                qv = q_v[d, pl.ds(soff, _F)]
                return tuple(
                    accs[f] + qv * plsc.load_gather(k_v, [idxs[f] + drow])
                    for f in range(_F))

            accs = lax.fori_loop(
                0, _D, score_d,
                tuple(jnp.zeros((_F,), jnp.float32) for _ in range(_F)))
            scores = [a * _SCALE for a in accs]
            sel = []
            for f in range(_F):
                dist = jnp.abs(srow + f - qpos).astype(jnp.float32)
                sel.append(scores[f] + _ALPHA * jnp.exp(dist * (-1.0 / _TAU)))
            rank = [jnp.zeros((_F,), jnp.float32) for _ in range(_F)]
            for lo in range(_F):
                for hi in range(lo + 1, _F):
                    ge = jnp.where(sel[lo] >= sel[hi], 1.0, 0.0)
                    rank[hi] = rank[hi] + ge
                    rank[lo] = rank[lo] + (1.0 - ge)
            mloc = scores[0]
            for f in range(1, _F):
                mloc = jnp.maximum(mloc, scores[f])
            es = [jnp.where(rank[f] < float(_KK),
                            jnp.exp(scores[f] - mloc), 0.0)
                  for f in range(_F)]
            z = es[0]
            for f in range(1, _F):
                z = z + es[f]
            mg = mz_v[0, pl.ds(soff, _F)]
            zg = mz_v[1, pl.ds(soff, _F)]
            mfull = jnp.maximum(mloc, mg)
            clc = jnp.exp(mloc - mfull)
            cgl = jnp.exp(mg - mfull)
            inv = 1.0 / (clc * z + cgl * zg)
            ws = [e * (clc * inv) for e in es]
            bg = cgl * inv

            def out_d(d, _o):
                drow = d * _CH
                acc = bg * og_v[d, pl.ds(soff, _F)]
                for f in range(_F):
                    acc = acc + ws[f] * plsc.load_gather(v_v, [idxs[f] + drow])
                out_v[d, pl.ds(soff, _F)] = acc
                return 0

            lax.fori_loop(0, _D, out_d, 0)
            return 0

        lax.fori_loop(0, _C // _F, group_body, 0)
        pltpu.sync_copy(out_v, outc.at[head, c_abs])
        return 0

    lax.fori_loop(0, _NCH // 2, chunk_body, 0)


def _round_bf16(x):
    u = jax.lax.bitcast_convert_type(x, jnp.uint32)
    bias = jnp.uint32(0x7FFF) + ((u >> 16) & jnp.uint32(1))
    return jax.lax.bitcast_convert_type((u + bias) & jnp.uint32(0xFFFF0000),
                                        jnp.float32)


def kernel(q, k, v):
    q3, k3, v3 = q[0], k[0], v[0]
    og, mz, qT, kT, vT = _globals_stage(k3, q3, v3)
    qc = qT.reshape(_H, _D, _NCH, _C).transpose(0, 2, 1, 3)
    kc = jnp.stack([kT[:, :, kb:kb + _CH] for kb in _KBS],
                   axis=1).reshape(_H, _NCH, _D * _CH)
    vc = jnp.stack([vT[:, :, kb:kb + _CH] for kb in _KBS],
                   axis=1).reshape(_H, _NCH, _D * _CH)
    ogc = og.reshape(_H, _D, _NCH, _C).transpose(0, 2, 1, 3)
    mzc = mz.reshape(_H, 2, _NCH, _C).transpose(0, 2, 1, 3)
    outc = _local_sc(qc, kc, vc, ogc, mzc)
    outT = outc.transpose(0, 2, 1, 3).reshape(_H, _D, _S)
    return outT.transpose(0, 2, 1)[None]
